# baseline (speedup 1.0000x reference)
import functools

import jax
import jax.numpy as jnp
from jax import lax
from jax.experimental import pallas as pl
from jax.experimental.pallas import tpu as pltpu
from jax.experimental.pallas import tpu_sc as plsc

N_FEAT = 26
VOCAB = 100001
EMB = 16
B = 1024
L = 200
H = 128
D = N_FEAT * EMB
TOP = 32

NC = 2
NS = 16
NW = NC * NS
N_TOT = L * B * N_FEAT
ROWS_PER_W = N_TOT // NW
KK = 10
CSZ = KK * 128
N_CHUNK = ROWS_PER_W // CSZ


def _sc_gather_body(table_hbm, idx_hbm, dst_hbm, out_hbm,
                    idx_v, dst_v, rows_v, sem):
    wid = lax.axis_index("s") * NC + lax.axis_index("c")

    def chunk(cj, carry):
        pltpu.sync_copy(idx_hbm.at[wid, cj], idx_v)
        pltpu.sync_copy(dst_hbm.at[wid, cj], dst_v)
        handles = []
        for jj in range(KK):
            handles.append(pltpu.async_copy(
                table_hbm.at[idx_v.at[jj]],
                rows_v.at[pl.ds(jj * 128, 128)], sem))
        for h in handles:
            h.wait()
        handles = []
        for jj in range(KK):
            handles.append(pltpu.async_copy(
                rows_v.at[pl.ds(jj * 128, 128)],
                out_hbm.at[dst_v.at[jj]], sem))
        for h in handles:
            h.wait()
        return carry

    lax.fori_loop(0, N_CHUNK, chunk, 0)


def _sc_gather(flat_table, idx_w, dst_w):
    mesh = plsc.VectorSubcoreMesh(core_axis_name="c", subcore_axis_name="s")
    k = functools.partial(
        pl.kernel,
        mesh=mesh,
        out_type=jax.ShapeDtypeStruct((N_TOT, EMB), jnp.float32),
        scratch_types=[
            pltpu.VMEM((KK, 128), jnp.int32),
            pltpu.VMEM((KK, 128), jnp.int32),
            pltpu.VMEM((CSZ, EMB), jnp.float32),
            pltpu.SemaphoreType.DMA,
        ],
        compiler_params=pltpu.CompilerParams(use_tc_tiling_on_sc=False),
    )(_sc_gather_body)
    return k(flat_table, idx_w, dst_w)


def _rnn_body(xf_ref, xb_ref, wihf, whhf, bihf, bhhf,
              wihb, whhb, bihb, bhhb, w1t, b1, w2t, b2,
              out_ref, hf, hb, mxf, mxb, smf, smb):
    t = pl.program_id(0)

    @pl.when(t == 0)
    def _init():
        z = jnp.zeros((B, H), jnp.float32)
        ninf = jnp.full((B, H), -jnp.inf, jnp.float32)
        hf[...] = z
        hb[...] = z
        smf[...] = z
        smb[...] = z
        mxf[...] = ninf
        mxb[...] = ninf

    def gru_step(x, h, wih, whh, bih, bhh):
        gi = jnp.dot(x, wih[...], preferred_element_type=jnp.float32) + bih[...]
        gh = jnp.dot(h, whh[...], preferred_element_type=jnp.float32) + bhh[...]
        i_r, i_z, i_n = gi[:, :H], gi[:, H:2 * H], gi[:, 2 * H:]
        h_r, h_z, h_n = gh[:, :H], gh[:, H:2 * H], gh[:, 2 * H:]
        r = jax.nn.sigmoid(i_r + h_r)
        z = jax.nn.sigmoid(i_z + h_z)
        n = jnp.tanh(i_n + r * h_n)
        return (1.0 - z) * n + z * h

    hf_new = gru_step(xf_ref[0], hf[...], wihf, whhf, bihf, bhhf)
    hb_new = gru_step(xb_ref[0], hb[...], wihb, whhb, bihb, bhhb)
    hf[...] = hf_new
    hb[...] = hb_new
    mxf[...] = jnp.maximum(mxf[...], hf_new)
    mxb[...] = jnp.maximum(mxb[...], hb_new)
    smf[...] = smf[...] + hf_new
    smb[...] = smb[...] + hb_new

    @pl.when(t == L - 1)
    def _final():
        inv_l = jnp.float32(1.0 / L)
        combined = jnp.concatenate(
            [mxf[...], mxb[...], smf[...] * inv_l, smb[...] * inv_l], axis=1)
        h1 = jnp.maximum(
            jnp.dot(combined, w1t[...], preferred_element_type=jnp.float32)
            + b1[...], 0.0)
        logit = jnp.dot(h1, w2t[...], preferred_element_type=jnp.float32) + b2[...]
        out_ref[...] = logit


def _rnn_call(x, wihf_t, whhf_t, bihf, bhhf, wihb_t, whhb_t, bihb, bhhb,
              w1t, b1, w2t, b2, interpret=False):
    full = lambda s: pl.BlockSpec(s, lambda t: (0,) * len(s))
    return pl.pallas_call(
        _rnn_body,
        grid=(L,),
        in_specs=[
            pl.BlockSpec((1, B, D), lambda t: (t, 0, 0)),
            pl.BlockSpec((1, B, D), lambda t: (L - 1 - t, 0, 0)),
            full((D, 3 * H)), full((H, 3 * H)), full((1, 3 * H)), full((1, 3 * H)),
            full((D, 3 * H)), full((H, 3 * H)), full((1, 3 * H)), full((1, 3 * H)),
            full((4 * H, TOP)), full((1, TOP)), full((TOP, 1)), full((1, 1)),
        ],
        out_specs=pl.BlockSpec((B, 1), lambda t: (0, 0)),
        out_shape=jax.ShapeDtypeStruct((B, 1), jnp.float32),
        scratch_shapes=[pltpu.VMEM((B, H), jnp.float32)] * 6,
        compiler_params=pltpu.CompilerParams(
            dimension_semantics=("arbitrary",)),
        interpret=interpret,
    )(x, x, wihf_t, whhf_t, bihf, bhhf, wihb_t, whhb_t, bihb, bhhb,
      w1t, b1, w2t, b2)


def kernel(transactions_cat_features, emb_tables, W_ih_f, W_hh_f, b_ih_f,
           b_hh_f, W_ih_b, W_hh_b, b_ih_b, b_hh_b, W1, b1, W2, b2):
    flat_table = emb_tables.reshape(N_FEAT * VOCAB, EMB)
    f_ax = jnp.arange(N_FEAT, dtype=jnp.int32)[:, None, None]
    b_ax = jnp.arange(B, dtype=jnp.int32)[None, :, None]
    t_ax = jnp.arange(L, dtype=jnp.int32)[None, None, :]
    idx_nat = transactions_cat_features + f_ax * VOCAB
    dst_nat = t_ax * (B * N_FEAT) + b_ax * N_FEAT + f_ax
    idx_w = idx_nat.reshape(NW, N_CHUNK, KK, 128)
    dst_w = dst_nat.reshape(NW, N_CHUNK, KK, 128)

    x = _sc_gather(flat_table, idx_w, dst_w).reshape(L, B, D)

    logit = _rnn_call(
        x,
        W_ih_f.T, W_hh_f.T, b_ih_f.reshape(1, -1), b_hh_f.reshape(1, -1),
        W_ih_b.T, W_hh_b.T, b_ih_b.reshape(1, -1), b_hh_b.reshape(1, -1),
        W1.T, b1.reshape(1, -1), W2.T, b2.reshape(1, -1))
    return logit


# speedup vs baseline: 1.0001x; 1.0001x over previous
import functools

import jax
import jax.numpy as jnp
from jax import lax
from jax.experimental import pallas as pl
from jax.experimental.pallas import tpu as pltpu
from jax.experimental.pallas import tpu_sc as plsc

N_FEAT = 26
VOCAB = 100001
EMB = 16
B = 1024
L = 200
H = 128
D = N_FEAT * EMB
TOP = 32

NC = 2
NS = 16
NW = NC * NS
N_TOT = L * B * N_FEAT
ROWS_PER_W = N_TOT // NW
KK = 10
CSZ = KK * 128
N_CHUNK = ROWS_PER_W // CSZ


def _sc_gather_body(table_hbm, idx_hbm, dst_hbm, out_hbm,
                    idx_v, dst_v, rows_v, sem):
    wid = lax.axis_index("s") * NC + lax.axis_index("c")
    row0 = wid * (ROWS_PER_W // 128)

    def chunk(cj, carry):
        pltpu.sync_copy(idx_hbm.at[pl.ds(row0 + cj * KK, KK)], idx_v)
        pltpu.sync_copy(dst_hbm.at[pl.ds(row0 + cj * KK, KK)], dst_v)
        handles = []
        for jj in range(KK):
            handles.append(pltpu.async_copy(
                table_hbm.at[idx_v.at[jj]],
                rows_v.at[pl.ds(jj * 128, 128)], sem))
        for h in handles:
            h.wait()
        handles = []
        for jj in range(KK):
            handles.append(pltpu.async_copy(
                rows_v.at[pl.ds(jj * 128, 128)],
                out_hbm.at[dst_v.at[jj]], sem))
        for h in handles:
            h.wait()
        return carry

    lax.fori_loop(0, N_CHUNK, chunk, 0)


def _sc_gather(flat_table, idx_w, dst_w):
    mesh = plsc.VectorSubcoreMesh(core_axis_name="c", subcore_axis_name="s")
    k = functools.partial(
        pl.kernel,
        mesh=mesh,
        out_type=jax.ShapeDtypeStruct((N_TOT, EMB), jnp.float32),
        scratch_types=[
            pltpu.VMEM((KK, 128), jnp.int32),
            pltpu.VMEM((KK, 128), jnp.int32),
            pltpu.VMEM((CSZ, EMB), jnp.float32),
            pltpu.SemaphoreType.DMA,
        ],
        compiler_params=pltpu.CompilerParams(use_tc_tiling_on_sc=False),
    )(_sc_gather_body)
    return k(flat_table, idx_w, dst_w)


def _rnn_body(xf_ref, xb_ref, wihf, whhf, bihf, bhhf,
              wihb, whhb, bihb, bhhb, w1t, b1, w2t, b2,
              out_ref, hf, hb, mxf, mxb, smf, smb):
    t = pl.program_id(0)

    @pl.when(t == 0)
    def _init():
        z = jnp.zeros((B, H), jnp.float32)
        ninf = jnp.full((B, H), -jnp.inf, jnp.float32)
        hf[...] = z
        hb[...] = z
        smf[...] = z
        smb[...] = z
        mxf[...] = ninf
        mxb[...] = ninf

    def gru_step(x, h, wih, whh, bih, bhh):
        gi = jnp.dot(x, wih[...], preferred_element_type=jnp.float32) + bih[...]
        gh = jnp.dot(h, whh[...], preferred_element_type=jnp.float32) + bhh[...]
        i_r, i_z, i_n = gi[:, :H], gi[:, H:2 * H], gi[:, 2 * H:]
        h_r, h_z, h_n = gh[:, :H], gh[:, H:2 * H], gh[:, 2 * H:]
        r = jax.nn.sigmoid(i_r + h_r)
        z = jax.nn.sigmoid(i_z + h_z)
        n = jnp.tanh(i_n + r * h_n)
        return (1.0 - z) * n + z * h

    hf_new = gru_step(xf_ref[0], hf[...], wihf, whhf, bihf, bhhf)
    hb_new = gru_step(xb_ref[0], hb[...], wihb, whhb, bihb, bhhb)
    hf[...] = hf_new
    hb[...] = hb_new
    mxf[...] = jnp.maximum(mxf[...], hf_new)
    mxb[...] = jnp.maximum(mxb[...], hb_new)
    smf[...] = smf[...] + hf_new
    smb[...] = smb[...] + hb_new

    @pl.when(t == L - 1)
    def _final():
        inv_l = jnp.float32(1.0 / L)
        combined = jnp.concatenate(
            [mxf[...], mxb[...], smf[...] * inv_l, smb[...] * inv_l], axis=1)
        h1 = jnp.maximum(
            jnp.dot(combined, w1t[...], preferred_element_type=jnp.float32)
            + b1[...], 0.0)
        logit = jnp.dot(h1, w2t[...], preferred_element_type=jnp.float32) + b2[...]
        out_ref[...] = logit


def _rnn_call(x, wihf_t, whhf_t, bihf, bhhf, wihb_t, whhb_t, bihb, bhhb,
              w1t, b1, w2t, b2, interpret=False):
    full = lambda s: pl.BlockSpec(s, lambda t: (0,) * len(s))
    return pl.pallas_call(
        _rnn_body,
        grid=(L,),
        in_specs=[
            pl.BlockSpec((1, B, D), lambda t: (t, 0, 0)),
            pl.BlockSpec((1, B, D), lambda t: (L - 1 - t, 0, 0)),
            full((D, 3 * H)), full((H, 3 * H)), full((1, 3 * H)), full((1, 3 * H)),
            full((D, 3 * H)), full((H, 3 * H)), full((1, 3 * H)), full((1, 3 * H)),
            full((4 * H, TOP)), full((1, TOP)), full((TOP, 1)), full((1, 1)),
        ],
        out_specs=pl.BlockSpec((B, 1), lambda t: (0, 0)),
        out_shape=jax.ShapeDtypeStruct((B, 1), jnp.float32),
        scratch_shapes=[pltpu.VMEM((B, H), jnp.float32)] * 6,
        compiler_params=pltpu.CompilerParams(
            dimension_semantics=("arbitrary",)),
        interpret=interpret,
    )(x, x, wihf_t, whhf_t, bihf, bhhf, wihb_t, whhb_t, bihb, bhhb,
      w1t, b1, w2t, b2)


def kernel(transactions_cat_features, emb_tables, W_ih_f, W_hh_f, b_ih_f,
           b_hh_f, W_ih_b, W_hh_b, b_ih_b, b_hh_b, W1, b1, W2, b2):
    flat_table = emb_tables.reshape(N_FEAT * VOCAB, EMB)
    f_ax = jnp.arange(N_FEAT, dtype=jnp.int32)[:, None, None]
    b_ax = jnp.arange(B, dtype=jnp.int32)[None, :, None]
    t_ax = jnp.arange(L, dtype=jnp.int32)[None, None, :]
    idx_nat = transactions_cat_features + f_ax * VOCAB
    dst_nat = t_ax * (B * N_FEAT) + b_ax * N_FEAT + f_ax
    idx_w = idx_nat.reshape(N_TOT // 128, 128)
    dst_w = jnp.broadcast_to(dst_nat, idx_nat.shape).reshape(N_TOT // 128, 128)

    x = _sc_gather(flat_table, idx_w, dst_w).reshape(L, B, D)

    logit = _rnn_call(
        x,
        W_ih_f.T, W_hh_f.T, b_ih_f.reshape(1, -1), b_hh_f.reshape(1, -1),
        W_ih_b.T, W_hh_b.T, b_ih_b.reshape(1, -1), b_hh_b.reshape(1, -1),
        W1.T, b1.reshape(1, -1), W2.T, b2.reshape(1, -1))
    return logit


# speedup vs baseline: 2.4356x; 2.4353x over previous
import functools

import jax
import jax.numpy as jnp
from jax import lax
from jax.experimental import pallas as pl
from jax.experimental.pallas import tpu as pltpu
from jax.experimental.pallas import tpu_sc as plsc

N_FEAT = 26
VOCAB = 100001
VOCAB_P = 100032
EMB = 16
B = 1024
L = 200
H = 128
D = N_FEAT * EMB
TOP = 32

NC = 2
NS = 16
NW = NC * NS
N_TOT = L * B * N_FEAT
ROWS_PER_W = N_TOT // NW
KK = 10
CSZ = KK * 128
N_CHUNK = ROWS_PER_W // CSZ


def _sc_gather_body(table_hbm, idx_hbm, dst_hbm, out_hbm,
                    idx_v, dst_v, rows_v, sem):
    wid = lax.axis_index("s") * NC + lax.axis_index("c")
    row0 = wid * (ROWS_PER_W // 128)

    def chunk(cj, carry):
        pltpu.sync_copy(idx_hbm.at[pl.ds(row0 + cj * KK, KK)], idx_v)
        pltpu.sync_copy(dst_hbm.at[pl.ds(row0 + cj * KK, KK)], dst_v)
        handles = []
        for jj in range(KK):
            handles.append(pltpu.async_copy(
                table_hbm.at[idx_v.at[jj]],
                rows_v.at[pl.ds(jj * 128, 128)], sem))
        for h in handles:
            h.wait()
        handles = []
        for jj in range(KK):
            handles.append(pltpu.async_copy(
                rows_v.at[pl.ds(jj * 128, 128)],
                out_hbm.at[dst_v.at[jj]], sem))
        for h in handles:
            h.wait()
        return carry

    lax.fori_loop(0, N_CHUNK, chunk, 0)


def _sc_gather(flat_table, idx_w, dst_w):
    mesh = plsc.VectorSubcoreMesh(core_axis_name="c", subcore_axis_name="s")
    k = functools.partial(
        pl.kernel,
        mesh=mesh,
        out_type=jax.ShapeDtypeStruct((N_TOT, EMB), jnp.float32),
        scratch_types=[
            pltpu.VMEM((KK, 128), jnp.int32),
            pltpu.VMEM((KK, 128), jnp.int32),
            pltpu.VMEM((CSZ, EMB), jnp.float32),
            pltpu.SemaphoreType.DMA,
        ],
        compiler_params=pltpu.CompilerParams(use_tc_tiling_on_sc=False),
    )(_sc_gather_body)
    return k(flat_table, idx_w, dst_w)


def _rnn_body(xf_ref, xb_ref, wihf, whhf, bihf, bhhf,
              wihb, whhb, bihb, bhhb, w1t, b1, w2t, b2,
              out_ref, hf, hb, mxf, mxb, smf, smb):
    t = pl.program_id(0)

    @pl.when(t == 0)
    def _init():
        z = jnp.zeros((B, H), jnp.float32)
        ninf = jnp.full((B, H), -jnp.inf, jnp.float32)
        hf[...] = z
        hb[...] = z
        smf[...] = z
        smb[...] = z
        mxf[...] = ninf
        mxb[...] = ninf

    def gru_step(x, h, wih, whh, bih, bhh):
        gi = jnp.dot(x, wih[...], preferred_element_type=jnp.float32) + bih[...]
        gh = jnp.dot(h, whh[...], preferred_element_type=jnp.float32) + bhh[...]
        i_r, i_z, i_n = gi[:, :H], gi[:, H:2 * H], gi[:, 2 * H:]
        h_r, h_z, h_n = gh[:, :H], gh[:, H:2 * H], gh[:, 2 * H:]
        r = jax.nn.sigmoid(i_r + h_r)
        z = jax.nn.sigmoid(i_z + h_z)
        n = jnp.tanh(i_n + r * h_n)
        return (1.0 - z) * n + z * h

    hf_new = gru_step(xf_ref[0], hf[...], wihf, whhf, bihf, bhhf)
    hb_new = gru_step(xb_ref[0], hb[...], wihb, whhb, bihb, bhhb)
    hf[...] = hf_new
    hb[...] = hb_new
    mxf[...] = jnp.maximum(mxf[...], hf_new)
    mxb[...] = jnp.maximum(mxb[...], hb_new)
    smf[...] = smf[...] + hf_new
    smb[...] = smb[...] + hb_new

    @pl.when(t == L - 1)
    def _final():
        inv_l = jnp.float32(1.0 / L)
        combined = jnp.concatenate(
            [mxf[...], mxb[...], smf[...] * inv_l, smb[...] * inv_l], axis=1)
        h1 = jnp.maximum(
            jnp.dot(combined, w1t[...], preferred_element_type=jnp.float32)
            + b1[...], 0.0)
        logit = jnp.dot(h1, w2t[...], preferred_element_type=jnp.float32) + b2[...]
        out_ref[...] = logit


def _rnn_call(x, wihf_t, whhf_t, bihf, bhhf, wihb_t, whhb_t, bihb, bhhb,
              w1t, b1, w2t, b2, interpret=False):
    full = lambda s: pl.BlockSpec(s, lambda t: (0,) * len(s))
    return pl.pallas_call(
        _rnn_body,
        grid=(L,),
        in_specs=[
            pl.BlockSpec((1, B, D), lambda t: (t, 0, 0)),
            pl.BlockSpec((1, B, D), lambda t: (L - 1 - t, 0, 0)),
            full((D, 3 * H)), full((H, 3 * H)), full((1, 3 * H)), full((1, 3 * H)),
            full((D, 3 * H)), full((H, 3 * H)), full((1, 3 * H)), full((1, 3 * H)),
            full((4 * H, TOP)), full((1, TOP)), full((TOP, 1)), full((1, 1)),
        ],
        out_specs=pl.BlockSpec((B, 1), lambda t: (0, 0)),
        out_shape=jax.ShapeDtypeStruct((B, 1), jnp.float32),
        scratch_shapes=[pltpu.VMEM((B, H), jnp.float32)] * 6,
        compiler_params=pltpu.CompilerParams(
            dimension_semantics=("arbitrary",)),
        interpret=interpret,
    )(x, x, wihf_t, whhf_t, bihf, bhhf, wihb_t, whhb_t, bihb, bhhb,
      w1t, b1, w2t, b2)


def kernel(transactions_cat_features, emb_tables, W_ih_f, W_hh_f, b_ih_f,
           b_hh_f, W_ih_b, W_hh_b, b_ih_b, b_hh_b, W1, b1, W2, b2):
    flat_table = jnp.pad(
        emb_tables, ((0, 0), (0, VOCAB_P - VOCAB), (0, 0))
    ).reshape(N_FEAT * VOCAB_P, EMB)
    f_ax = jnp.arange(N_FEAT, dtype=jnp.int32)[:, None, None]
    b_ax = jnp.arange(B, dtype=jnp.int32)[None, :, None]
    t_ax = jnp.arange(L, dtype=jnp.int32)[None, None, :]
    idx_nat = transactions_cat_features + f_ax * VOCAB_P
    dst_nat = t_ax * (B * N_FEAT) + b_ax * N_FEAT + f_ax
    idx_w = idx_nat.reshape(N_TOT // 128, 128)
    dst_w = jnp.broadcast_to(dst_nat, idx_nat.shape).reshape(N_TOT // 128, 128)

    x = _sc_gather(flat_table, idx_w, dst_w).reshape(L, B, D)

    logit = _rnn_call(
        x,
        W_ih_f.T, W_hh_f.T, b_ih_f.reshape(1, -1), b_hh_f.reshape(1, -1),
        W_ih_b.T, W_hh_b.T, b_ih_b.reshape(1, -1), b_hh_b.reshape(1, -1),
        W1.T, b1.reshape(1, -1), W2.T, b2.reshape(1, -1))
    return logit


# speedup vs baseline: 2.4490x; 1.0055x over previous
import functools

import jax
import jax.numpy as jnp
from jax import lax
from jax.experimental import pallas as pl
from jax.experimental.pallas import tpu as pltpu
from jax.experimental.pallas import tpu_sc as plsc

N_FEAT = 26
VOCAB = 100001
VOCAB_P = 100032
EMB = 16
B = 1024
L = 200
H = 128
D = N_FEAT * EMB
TOP = 32
D_P = 512

NC = 2
NS = 16
NW = NC * NS
N_TOT = L * B * N_FEAT
SLOT = 32
N_OUT = L * B * SLOT
ROWS_PER_W = N_TOT // NW
KK = 10
CSZ = KK * 128
N_CHUNK = ROWS_PER_W // CSZ
CHUNKS_PER_F = (B * L) // CSZ


def _sc_gather_body(table_hbm, idx_hbm, dst_hbm, out_hbm,
                    idx_v, dst_v, rows_v, sem):
    wid = lax.axis_index("s") * NC + lax.axis_index("c")
    row0 = wid * (ROWS_PER_W // 128)

    def chunk(cj, carry):
        f = (wid * N_CHUNK + cj) // CHUNKS_PER_F
        pltpu.sync_copy(idx_hbm.at[pl.ds(row0 + cj * KK, KK)], idx_v)
        pltpu.sync_copy(dst_hbm.at[pl.ds(row0 + cj * KK, KK)], dst_v)
        handles = []
        for jj in range(KK):
            handles.append(pltpu.async_copy(
                table_hbm.at[f].at[idx_v.at[jj]],
                rows_v.at[pl.ds(jj * 128, 128)], sem))
        for h in handles:
            h.wait()
        handles = []
        for jj in range(KK):
            handles.append(pltpu.async_copy(
                rows_v.at[pl.ds(jj * 128, 128)],
                out_hbm.at[dst_v.at[jj]], sem))
        for h in handles:
            h.wait()
        return carry

    lax.fori_loop(0, N_CHUNK, chunk, 0)


def _sc_gather(flat_table, idx_w, dst_w):
    mesh = plsc.VectorSubcoreMesh(core_axis_name="c", subcore_axis_name="s")
    k = functools.partial(
        pl.kernel,
        mesh=mesh,
        out_type=jax.ShapeDtypeStruct((N_OUT, EMB), jnp.float32),
        scratch_types=[
            pltpu.VMEM((KK, 128), jnp.int32),
            pltpu.VMEM((KK, 128), jnp.int32),
            pltpu.VMEM((CSZ, EMB), jnp.float32),
            pltpu.SemaphoreType.DMA,
        ],
        compiler_params=pltpu.CompilerParams(use_tc_tiling_on_sc=False),
    )(_sc_gather_body)
    return k(flat_table, idx_w, dst_w)


def _rnn_body(xf_ref, xb_ref, wihf, whhf, bihf, bhhf,
              wihb, whhb, bihb, bhhb, w1t, b1, w2t, b2,
              out_ref, hf, hb, mxf, mxb, smf, smb):
    t = pl.program_id(0)

    @pl.when(t == 0)
    def _init():
        z = jnp.zeros((B, H), jnp.float32)
        ninf = jnp.full((B, H), -jnp.inf, jnp.float32)
        hf[...] = z
        hb[...] = z
        smf[...] = z
        smb[...] = z
        mxf[...] = ninf
        mxb[...] = ninf

    def gru_step(x, h, wih, whh, bih, bhh):
        gi = jnp.dot(x, wih[...], preferred_element_type=jnp.float32) + bih[...]
        gh = jnp.dot(h, whh[...], preferred_element_type=jnp.float32) + bhh[...]
        i_r, i_z, i_n = gi[:, :H], gi[:, H:2 * H], gi[:, 2 * H:]
        h_r, h_z, h_n = gh[:, :H], gh[:, H:2 * H], gh[:, 2 * H:]
        r = jax.nn.sigmoid(i_r + h_r)
        z = jax.nn.sigmoid(i_z + h_z)
        n = jnp.tanh(i_n + r * h_n)
        return (1.0 - z) * n + z * h

    lane = lax.broadcasted_iota(jnp.int32, (B, D_P), 1)
    xf = jnp.where(lane < D, xf_ref[0], 0.0)
    xb = jnp.where(lane < D, xb_ref[0], 0.0)
    hf_new = gru_step(xf, hf[...], wihf, whhf, bihf, bhhf)
    hb_new = gru_step(xb, hb[...], wihb, whhb, bihb, bhhb)
    hf[...] = hf_new
    hb[...] = hb_new
    mxf[...] = jnp.maximum(mxf[...], hf_new)
    mxb[...] = jnp.maximum(mxb[...], hb_new)
    smf[...] = smf[...] + hf_new
    smb[...] = smb[...] + hb_new

    @pl.when(t == L - 1)
    def _final():
        inv_l = jnp.float32(1.0 / L)
        combined = jnp.concatenate(
            [mxf[...], mxb[...], smf[...] * inv_l, smb[...] * inv_l], axis=1)
        h1 = jnp.maximum(
            jnp.dot(combined, w1t[...], preferred_element_type=jnp.float32)
            + b1[...], 0.0)
        logit = jnp.dot(h1, w2t[...], preferred_element_type=jnp.float32) + b2[...]
        out_ref[...] = logit


def _rnn_call(x, wihf_t, whhf_t, bihf, bhhf, wihb_t, whhb_t, bihb, bhhb,
              w1t, b1, w2t, b2, interpret=False):
    full = lambda s: pl.BlockSpec(s, lambda t: (0,) * len(s))
    return pl.pallas_call(
        _rnn_body,
        grid=(L,),
        in_specs=[
            pl.BlockSpec((1, B, D_P), lambda t: (t, 0, 0)),
            pl.BlockSpec((1, B, D_P), lambda t: (L - 1 - t, 0, 0)),
            full((D_P, 3 * H)), full((H, 3 * H)), full((1, 3 * H)), full((1, 3 * H)),
            full((D_P, 3 * H)), full((H, 3 * H)), full((1, 3 * H)), full((1, 3 * H)),
            full((4 * H, TOP)), full((1, TOP)), full((TOP, 1)), full((1, 1)),
        ],
        out_specs=pl.BlockSpec((B, 1), lambda t: (0, 0)),
        out_shape=jax.ShapeDtypeStruct((B, 1), jnp.float32),
        scratch_shapes=[pltpu.VMEM((B, H), jnp.float32)] * 6,
        compiler_params=pltpu.CompilerParams(
            dimension_semantics=("arbitrary",)),
        interpret=interpret,
    )(x, x, wihf_t, whhf_t, bihf, bhhf, wihb_t, whhb_t, bihb, bhhb,
      w1t, b1, w2t, b2)


def kernel(transactions_cat_features, emb_tables, W_ih_f, W_hh_f, b_ih_f,
           b_hh_f, W_ih_b, W_hh_b, b_ih_b, b_hh_b, W1, b1, W2, b2):
    table_p = jnp.pad(emb_tables, ((0, 0), (0, VOCAB_P - VOCAB), (0, 0)))
    f_ax = jnp.arange(N_FEAT, dtype=jnp.int32)[:, None, None]
    b_ax = jnp.arange(B, dtype=jnp.int32)[None, :, None]
    t_ax = jnp.arange(L, dtype=jnp.int32)[None, None, :]
    dst_nat = t_ax * (B * SLOT) + b_ax * SLOT + f_ax
    idx_w = transactions_cat_features.reshape(N_TOT // 128, 128)
    dst_w = jnp.broadcast_to(
        dst_nat, transactions_cat_features.shape).reshape(N_TOT // 128, 128)

    x = _sc_gather(table_p, idx_w, dst_w).reshape(L, B, D_P)

    wpad = lambda w: jnp.pad(w.T, ((0, D_P - D), (0, 0)))
    logit = _rnn_call(
        x,
        wpad(W_ih_f), W_hh_f.T, b_ih_f.reshape(1, -1), b_hh_f.reshape(1, -1),
        wpad(W_ih_b), W_hh_b.T, b_ih_b.reshape(1, -1), b_hh_b.reshape(1, -1),
        W1.T, b1.reshape(1, -1), W2.T, b2.reshape(1, -1))
    return logit
